# Initial kernel scaffold; baseline (speedup 1.0000x reference)
import jax
import jax.numpy as jnp
from jax.experimental import pallas as pl


def kernel(feat, edge_index, etypes, weight):
    raise NotImplementedError("write your pallas kernel here")


# speedup vs baseline: 1.5794x; 1.5794x over previous
import functools

import jax
import jax.numpy as jnp
from jax import lax
from jax.experimental import pallas as pl
from jax.experimental.pallas import tpu as pltpu
from jax.experimental.pallas import tpu_sc as plsc

NC = 2
NS = 16
NW = NC * NS

B = 512
C = 256


def _gather_kernel(e_pad, n_nodes, d):
    nchunk = e_pad // C
    mesh = plsc.VectorSubcoreMesh(core_axis_name="c", subcore_axis_name="s")

    @functools.partial(
        pl.kernel,
        mesh=mesh,
        out_type=jax.ShapeDtypeStruct((e_pad, d), jnp.float32),
        scratch_types=[
            pltpu.VMEM((C,), jnp.int32),
            pltpu.VMEM((C, d), jnp.float32),
            pltpu.SemaphoreType.DMA,
        ],
    )
    def gather_k(src_hbm, feat_hbm, h_hbm, idx_v, rows_v, sem):
        wid = lax.axis_index("s") * NC + lax.axis_index("c")
        nch = (nchunk - wid + NW - 1) // NW

        def body(i, carry):
            base = (wid + i * NW) * C
            pltpu.sync_copy(src_hbm.at[pl.ds(base, C)], idx_v)
            pltpu.async_copy(feat_hbm.at[idx_v], rows_v, sem).wait()
            pltpu.sync_copy(rows_v, h_hbm.at[pl.ds(base, C)])
            return carry

        lax.fori_loop(0, nch, body, 0)

    return gather_k


def _scatter_kernel(e_pad, nn_pad, d):
    nchunk = e_pad // C
    rows_per_tile = nn_pad // NS
    mesh = plsc.VectorSubcoreMesh(core_axis_name="c", subcore_axis_name="s")

    @functools.partial(
        pl.kernel,
        mesh=mesh,
        out_type=jax.ShapeDtypeStruct((NC * nn_pad, d), jnp.float32),
        scratch_types=[
            pltpu.VMEM((C,), jnp.int32),
            pltpu.VMEM((C, d), jnp.float32),
            pltpu.VMEM_SHARED((nn_pad, d), jnp.float32),
            pltpu.SemaphoreType.DMA,
        ],
    )
    def scatter_k(dst_hbm, m_hbm, zeros_hbm, out_hbm, idx_v, rows_v, acc, sem):
        cid = lax.axis_index("c")
        sid = lax.axis_index("s")
        wid = sid * NC + cid
        pltpu.sync_copy(
            zeros_hbm.at[pl.ds(sid * rows_per_tile, rows_per_tile)],
            acc.at[pl.ds(sid * rows_per_tile, rows_per_tile)],
        )
        plsc.subcore_barrier()
        nch = (nchunk - wid + NW - 1) // NW

        def body(i, carry):
            base = (wid + i * NW) * C
            pltpu.sync_copy(dst_hbm.at[pl.ds(base, C)], idx_v)
            pltpu.sync_copy(m_hbm.at[pl.ds(base, C)], rows_v)
            pltpu.sync_copy(rows_v, acc.at[idx_v], add=True)
            return carry

        lax.fori_loop(0, nch, body, 0)
        plsc.subcore_barrier()
        pltpu.sync_copy(
            acc.at[pl.ds(sid * rows_per_tile, rows_per_tile)],
            out_hbm.at[pl.ds(cid * nn_pad + sid * rows_per_tile, rows_per_tile)],
        )

    return scatter_k


def kernel(feat, edge_index, etypes, weight):
    n_nodes, d_in = feat.shape
    num_rels, _, d_out = weight.shape
    n_edges = etypes.shape[0]

    nblk_max = (n_edges + B - 1) // B + num_rels
    e_pad = nblk_max * B
    nn_pad = ((n_nodes + 1 + NS * 8 - 1) // (NS * 8)) * (NS * 8)

    order = jnp.argsort(etypes)
    src_s = edge_index[0][order]
    dst_s = edge_index[1][order]
    ets_s = etypes[order]
    offs = jnp.searchsorted(ets_s, jnp.arange(num_rels + 1, dtype=jnp.int32),
                            side="left").astype(jnp.int32)
    counts = offs[1:] - offs[:-1]
    nblk = (counts + B - 1) // B
    blk_end = jnp.cumsum(nblk)
    blk_off = blk_end - nblk
    bids = jnp.arange(nblk_max, dtype=jnp.int32)
    brel = jnp.searchsorted(blk_end, bids, side="right").astype(jnp.int32)
    brel_c = jnp.minimum(brel, num_rels - 1)

    slot = jnp.arange(e_pad, dtype=jnp.int32)
    blk_of_slot = slot // B
    r_of_slot = brel_c[blk_of_slot]
    pos = (blk_of_slot - blk_off[r_of_slot]) * B + (slot % B)
    valid = (brel[blk_of_slot] < num_rels) & (pos < counts[r_of_slot])
    e_idx = jnp.clip(offs[r_of_slot] + pos, 0, n_edges - 1)
    src_pad = jnp.where(valid, src_s[e_idx], 0).astype(jnp.int32)
    dst_pad = jnp.where(valid, dst_s[e_idx], n_nodes).astype(jnp.int32)

    h = _gather_kernel(e_pad, n_nodes, d_in)(src_pad, feat)

    def mm_body(brel_ref, h_ref, w_ref, m_ref):
        m_ref[...] = lax.dot_general(
            h_ref[...], w_ref[0], (((1,), (0,)), ((), ())),
            preferred_element_type=jnp.float32)

    grid_spec = pltpu.PrefetchScalarGridSpec(
        num_scalar_prefetch=1,
        grid=(nblk_max,),
        in_specs=[
            pl.BlockSpec((B, d_in), lambda b, brel: (b, 0)),
            pl.BlockSpec((1, d_in, d_out), lambda b, brel: (brel[b], 0, 0)),
        ],
        out_specs=pl.BlockSpec((B, d_out), lambda b, brel: (b, 0)),
    )
    m = pl.pallas_call(
        mm_body,
        grid_spec=grid_spec,
        out_shape=jax.ShapeDtypeStruct((e_pad, d_out), jnp.float32),
    )(brel_c, h, weight)

    zeros = jnp.zeros((nn_pad, d_out), jnp.float32)
    partials = _scatter_kernel(e_pad, nn_pad, d_out)(dst_pad, m, zeros)
    partials = partials.reshape(NC, nn_pad, d_out)

    rows_blk = 1000

    def add_body(a_ref, b_ref, o_ref):
        o_ref[...] = a_ref[0] + b_ref[0]

    out = pl.pallas_call(
        add_body,
        grid=(n_nodes // rows_blk,),
        in_specs=[
            pl.BlockSpec((1, rows_blk, d_out), lambda i: (0, i, 0)),
            pl.BlockSpec((1, rows_blk, d_out), lambda i: (1, i, 0)),
        ],
        out_specs=pl.BlockSpec((rows_blk, d_out), lambda i: (i, 0)),
        out_shape=jax.ShapeDtypeStruct((n_nodes, d_out), jnp.float32),
    )(partials, partials)
    return out


# speedup vs baseline: 8.8293x; 5.5902x over previous
import functools

import jax
import jax.numpy as jnp
from jax import lax
from jax.experimental import pallas as pl
from jax.experimental.pallas import tpu as pltpu
from jax.experimental.pallas import tpu_sc as plsc

NC = 2
NS = 16
NW = NC * NS

B = 512
C = 128


def _sc_mesh():
    return plsc.VectorSubcoreMesh(core_axis_name="c", subcore_axis_name="s")


def _gather_kernel(ec, e_trash, d):
    nch = ec // C
    npw = nch // NW

    @functools.partial(
        pl.kernel,
        mesh=_sc_mesh(),
        out_type=jax.ShapeDtypeStruct((e_trash + C, d), jnp.float32),
        scratch_types=[
            pltpu.VMEM((C,), jnp.int32), pltpu.VMEM((C,), jnp.int32),
            pltpu.VMEM((C,), jnp.int32), pltpu.VMEM((C,), jnp.int32),
            pltpu.VMEM((2, C, d), jnp.float32),
            pltpu.SemaphoreType.DMA, pltpu.SemaphoreType.DMA,
            pltpu.SemaphoreType.DMA, pltpu.SemaphoreType.DMA,
            pltpu.SemaphoreType.DMA, pltpu.SemaphoreType.DMA,
        ],
    )
    def gather_k(src_hbm, slot_hbm, feat_hbm, h_hbm,
                 si0, si1, sl0, sl1, rows,
                 semi0, semi1, semg0, semg1, sems0, sems1):
        wid = lax.axis_index("s") * NC + lax.axis_index("c")
        si = (si0, si1)
        sl = (sl0, sl1)
        semi = (semi0, semi1)
        semg = (semg0, semg1)
        sems = (sems0, sems1)

        def load_idx(i, b):
            base = (wid + i * NW) * C
            pltpu.async_copy(src_hbm.at[pl.ds(base, C)], si[b], semi[b])
            pltpu.async_copy(slot_hbm.at[pl.ds(base, C)], sl[b], semi[b])

        def do_chunk(i, b, wait_prev, prefetch):
            nb = 1 - b
            if wait_prev:
                pltpu.make_async_copy(rows.at[nb], h_hbm.at[sl[nb]], sems[nb]).wait()
            if prefetch:
                load_idx(i + 1, nb)
            pltpu.make_async_copy(src_hbm.at[pl.ds(0, C)], si[b], semi[b]).wait()
            pltpu.make_async_copy(slot_hbm.at[pl.ds(0, C)], sl[b], semi[b]).wait()
            pltpu.async_copy(feat_hbm.at[si[b]], rows.at[b], semg[b]).wait()
            pltpu.async_copy(rows.at[b], h_hbm.at[sl[b]], sems[b])

        load_idx(0, 0)
        do_chunk(0, 0, False, True)
        do_chunk(1, 1, True, True)

        def body(g, carry):
            do_chunk(2 * g, 0, True, True)
            do_chunk(2 * g + 1, 1, True, True)
            return carry

        lax.fori_loop(1, npw // 2 - 1, body, 0)
        do_chunk(npw - 2, 0, True, True)
        do_chunk(npw - 1, 1, True, False)
        pltpu.make_async_copy(rows.at[1], h_hbm.at[sl[1]], sems[1]).wait()

    return gather_k


def _scatter_kernel(ec, e_pad, nn_pad, d):
    nch = ec // C
    npw = nch // NW
    rpt = nn_pad // NS

    @functools.partial(
        pl.kernel,
        mesh=_sc_mesh(),
        out_type=jax.ShapeDtypeStruct((NC * nn_pad, d), jnp.float32),
        scratch_types=[
            pltpu.VMEM((C,), jnp.int32), pltpu.VMEM((C,), jnp.int32),
            pltpu.VMEM((C,), jnp.int32), pltpu.VMEM((C,), jnp.int32),
            pltpu.VMEM((2, C, d), jnp.float32),
            pltpu.VMEM_SHARED((nn_pad, d), jnp.float32),
            pltpu.SemaphoreType.DMA, pltpu.SemaphoreType.DMA,
            pltpu.SemaphoreType.DMA, pltpu.SemaphoreType.DMA,
            pltpu.SemaphoreType.DMA, pltpu.SemaphoreType.DMA,
        ],
    )
    def scatter_k(slot_hbm, dst_hbm, m_hbm, zeros_hbm, out_hbm,
                  sl0, sl1, ds0, ds1, rows, acc,
                  semi0, semi1, semg0, semg1, sema0, sema1):
        cid = lax.axis_index("c")
        sid = lax.axis_index("s")
        wid = sid * NC + cid
        sl = (sl0, sl1)
        ds = (ds0, ds1)
        semi = (semi0, semi1)
        semg = (semg0, semg1)
        sema = (sema0, sema1)

        pltpu.sync_copy(zeros_hbm.at[pl.ds(sid * rpt, rpt)],
                        acc.at[pl.ds(sid * rpt, rpt)])
        plsc.subcore_barrier()

        def load_idx(i, b):
            base = (wid + i * NW) * C
            pltpu.async_copy(slot_hbm.at[pl.ds(base, C)], sl[b], semi[b])
            pltpu.async_copy(dst_hbm.at[pl.ds(base, C)], ds[b], semi[b])

        def do_chunk(i, b, wait_prev, prefetch):
            nb = 1 - b
            if wait_prev:
                pltpu.make_async_copy(rows.at[nb], acc.at[ds[nb]], sema[nb]).wait()
            if prefetch:
                load_idx(i + 1, nb)
            pltpu.make_async_copy(slot_hbm.at[pl.ds(0, C)], sl[b], semi[b]).wait()
            pltpu.make_async_copy(dst_hbm.at[pl.ds(0, C)], ds[b], semi[b]).wait()
            pltpu.async_copy(m_hbm.at[sl[b]], rows.at[b], semg[b]).wait()
            pltpu.async_copy(rows.at[b], acc.at[ds[b]], sema[b], add=True)

        load_idx(0, 0)
        do_chunk(0, 0, False, True)
        do_chunk(1, 1, True, True)

        def body(g, carry):
            do_chunk(2 * g, 0, True, True)
            do_chunk(2 * g + 1, 1, True, True)
            return carry

        lax.fori_loop(1, npw // 2 - 1, body, 0)
        do_chunk(npw - 2, 0, True, True)
        do_chunk(npw - 1, 1, True, False)
        pltpu.make_async_copy(rows.at[1], acc.at[ds[1]], sema[1]).wait()

        plsc.subcore_barrier()
        pltpu.sync_copy(acc.at[pl.ds(sid * rpt, rpt)],
                        out_hbm.at[pl.ds(cid * nn_pad + sid * rpt, rpt)])

    return scatter_k


def kernel(feat, edge_index, etypes, weight):
    n_nodes, d_in = feat.shape
    num_rels, _, d_out = weight.shape
    n_edges = etypes.shape[0]

    nblk_max = n_edges // B + num_rels
    e_pad = nblk_max * B
    nn_pad = ((n_nodes + 1 + NS * 8 - 1) // (NS * 8)) * (NS * 8)
    ec = ((n_edges + 2 * NW * C - 1) // (2 * NW * C)) * (2 * NW * C)

    ch = 128
    nchk = n_edges // ch
    r_ids = jnp.arange(num_rels, dtype=etypes.dtype)
    oh = (etypes.reshape(nchk, ch)[:, :, None] == r_ids).astype(jnp.bfloat16)
    tril = jnp.tril(jnp.ones((ch, ch), jnp.bfloat16))
    within = jnp.einsum("ij,cjr->cir", tril, oh,
                        preferred_element_type=jnp.bfloat16)
    totals = within[:, -1, :].astype(jnp.float32)
    g2 = (nchk + ch - 1) // ch
    t2 = jnp.pad(totals, ((0, g2 * ch - nchk), (0, 0)))
    w2 = jnp.einsum("ij,gjr->gir", tril.astype(jnp.float32),
                    t2.reshape(g2, ch, num_rels),
                    preferred_element_type=jnp.float32)
    tot2 = w2[:, -1, :]
    base2 = jnp.cumsum(tot2, axis=0) - tot2
    chunk_base = (base2[:, None, :] + w2 - t2.reshape(g2, ch, num_rels)
                  ).reshape(g2 * ch, num_rels)[:nchk]
    counts = (base2[-1] + tot2[-1]).astype(jnp.int32)
    nblk = (counts + B - 1) // B
    blk_end = jnp.cumsum(nblk)
    blk_offb = ((blk_end - nblk) * B).astype(jnp.float32)
    table = (chunk_base[:, None, :] + within.astype(jnp.float32)
             + blk_offb[None, None, :] - 1.0)
    slot = jnp.sum(table * oh.astype(jnp.float32), axis=-1
                   ).reshape(n_edges).astype(jnp.int32)

    bids = jnp.arange(nblk_max, dtype=jnp.int32)
    brel = jnp.minimum(
        jnp.searchsorted(blk_end, bids, side="right").astype(jnp.int32),
        num_rels - 1)

    pad = ec - n_edges
    src_g = jnp.concatenate([edge_index[0].astype(jnp.int32),
                             jnp.zeros((pad,), jnp.int32)])
    slot_g = jnp.concatenate([slot, jnp.full((pad,), e_pad, jnp.int32)])
    slot_s = jnp.concatenate([slot, jnp.zeros((pad,), jnp.int32)])
    dst_s = jnp.concatenate([edge_index[1].astype(jnp.int32),
                             jnp.full((pad,), n_nodes, jnp.int32)])

    h = _gather_kernel(ec, e_pad, d_in)(src_g, slot_g, feat)

    def mm_body(brel_ref, h_ref, w_ref, m_ref):
        m_ref[...] = lax.dot_general(
            h_ref[...], w_ref[0], (((1,), (0,)), ((), ())),
            preferred_element_type=jnp.float32)

    grid_spec = pltpu.PrefetchScalarGridSpec(
        num_scalar_prefetch=1,
        grid=(nblk_max,),
        in_specs=[
            pl.BlockSpec((B, d_in), lambda b, brel: (b, 0)),
            pl.BlockSpec((1, d_in, d_out), lambda b, brel: (brel[b], 0, 0)),
        ],
        out_specs=pl.BlockSpec((B, d_out), lambda b, brel: (b, 0)),
    )
    m = pl.pallas_call(
        mm_body,
        grid_spec=grid_spec,
        out_shape=jax.ShapeDtypeStruct((e_pad, d_out), jnp.float32),
    )(brel, h, weight)

    zeros = jnp.zeros((nn_pad, d_out), jnp.float32)
    partials = _scatter_kernel(ec, e_pad, nn_pad, d_out)(slot_s, dst_s, m, zeros)
    partials = partials.reshape(NC, nn_pad, d_out)

    rows_blk = 1000

    def add_body(a_ref, b_ref, o_ref):
        o_ref[...] = a_ref[0] + b_ref[0]

    out = pl.pallas_call(
        add_body,
        grid=(n_nodes // rows_blk,),
        in_specs=[
            pl.BlockSpec((1, rows_blk, d_out), lambda i: (0, i, 0)),
            pl.BlockSpec((1, rows_blk, d_out), lambda i: (1, i, 0)),
        ],
        out_specs=pl.BlockSpec((rows_blk, d_out), lambda i: (i, 0)),
        out_shape=jax.ShapeDtypeStruct((n_nodes, d_out), jnp.float32),
    )(partials, partials)
    return out
